# baseline (speedup 1.0000x reference)
import functools

import jax
import jax.numpy as jnp
from jax import lax
from jax.experimental import pallas as pl
from jax.experimental.pallas import tpu as pltpu
from jax.experimental.pallas import tpu_sc as plsc

B, T, N, F_IN = 256, 8, 30, 128
NF, C = 64, 10
NC, NS = 2, 16
NW = NC * NS
GPW = B // NW
L = 16
NEG = -1e30


GX = 32


def _pre_body(real_ref, sigs_ref, wconv_ref, adj_ref, xw_ref):
    adj_ref[...] = jnp.sum(sigs_ref[...], axis=1)
    x = real_ref[...].reshape(GX * N, F_IN)
    xw = jnp.dot(x, wconv_ref[...], preferred_element_type=jnp.float32,
                 precision=lax.Precision.HIGHEST)
    xw_ref[...] = (xw * (1.0 / T)).reshape(GX, N, NF)


def _tc_pre(real, graph_sigs, W_conv):
    return pl.pallas_call(
        _pre_body,
        grid=(B // GX,),
        in_specs=[pl.BlockSpec((GX, N, F_IN), lambda i: (i, 0, 0)),
                  pl.BlockSpec((GX, T, N, N), lambda i: (i, 0, 0, 0)),
                  pl.BlockSpec((F_IN, NF), lambda i: (0, 0))],
        out_specs=[pl.BlockSpec((GX, N, N), lambda i: (i, 0, 0)),
                   pl.BlockSpec((GX, N, NF), lambda i: (i, 0, 0))],
        out_shape=[jax.ShapeDtypeStruct((B, N, N), jnp.float32),
                   jax.ShapeDtypeStruct((B, N, NF), jnp.float32)],
    )(real, graph_sigs, W_conv)


_JBLOCKS = ((0, 8), (8, 8), (16, 8), (24, 6))


def _bfly(v, op):
    dnums = lax.GatherDimensionNumbers(offset_dims=(), collapsed_slice_dims=(0,),
                                       start_index_map=(0,))
    lane = lax.iota(jnp.int32, L)
    for s in (8, 4, 2, 1):
        perm = (lane ^ s)[:, None]
        g = lax.gather(v, perm, dnums, (1,),
                       mode=lax.GatherScatterMode.PROMISE_IN_BOUNDS)
        v = op(v, g)
    return v


def _sc_body(adj_hbm, xw_hbm, m_hbm, bconv_hbm, bias2_hbm, out_hbm,
             adj_v, xw_v, m_v, bconv_v, bias2_v, out_v):
    cid = lax.axis_index("c")
    sid = lax.axis_index("s")
    wid = sid * NC + cid

    pltpu.sync_copy(m_hbm, m_v)
    pltpu.sync_copy(bconv_hbm, bconv_v)
    pltpu.sync_copy(bias2_hbm, bias2_v)
    bconv_r = [bconv_v[pl.ds(fv * L, L)] for fv in range(NF // L)]
    bias2_r = bias2_v[...]

    def graph_body(k, _):
        g = wid * GPW + k
        pltpu.sync_copy(adj_hbm.at[g], adj_v)
        pltpu.sync_copy(xw_hbm.at[g], xw_v)

        accC = tuple(jnp.zeros((L,), jnp.float32) for _ in range(C))
        for (j0, J) in _JBLOCKS:
            def agg_body(i, accs, j0=j0, J=J):
                xwr = [xw_v[i, pl.ds(fv * L, L)] for fv in range(NF // L)]
                av = adj_v[i, pl.ds(j0, L)] if j0 + L <= N else \
                    adj_v[i, pl.ds(N - L, L)]
                out = []
                for jj in range(J):
                    a = av[jj] if j0 + L <= N else av[jj + j0 - (N - L)]
                    row = accs[jj]
                    out.append(tuple(row[fv] + a * xwr[fv]
                                     for fv in range(NF // L)))
                return tuple(out)

            init = tuple(tuple(jnp.zeros((L,), jnp.float32)
                               for _ in range(NF // L)) for _ in range(J))
            accs = lax.fori_loop(0, N, agg_body, init)
            accC = list(accC)
            for jj in range(J):
                for fv in range(NF // L):
                    h = jnp.maximum(accs[jj][fv] + bconv_r[fv], 0.0)
                    for c in range(C):
                        accC[c] = accC[c] + m_v[c, j0 + jj,
                                                pl.ds(fv * L, L)] * h
            accC = tuple(accC)

        lane = lax.iota(jnp.int32, L)
        lv = bias2_r
        for c in range(C):
            lv = lv + jnp.where(lane == c, _bfly(accC[c], jnp.add), 0.0)
        mxv = _bfly(lv, jnp.maximum)
        e = jnp.exp(lv - mxv)
        sv = _bfly(e, jnp.add)
        out_v[k, :] = e / sv
        return 0

    lax.fori_loop(0, GPW, graph_body, 0)
    pltpu.sync_copy(out_v, out_hbm.at[pl.ds(wid * GPW, GPW)])


@functools.lru_cache(maxsize=1)
def _sc_main():
    return pl.kernel(
        _sc_body,
        out_type=jax.ShapeDtypeStruct((B, L), jnp.float32),
        mesh=plsc.VectorSubcoreMesh(core_axis_name="c", subcore_axis_name="s",
                                    num_cores=NC, num_subcores=NS),
        scratch_types=[
            pltpu.VMEM((N, N), jnp.float32),
            pltpu.VMEM((N, NF), jnp.float32),
            pltpu.VMEM((C, N, NF), jnp.float32),
            pltpu.VMEM((NF,), jnp.float32),
            pltpu.VMEM((L,), jnp.float32),
            pltpu.VMEM((GPW, L), jnp.float32),
        ],
    )


def kernel(real, imag, graph_sigs, W_conv, b_conv, w_pool, b_pool, W_head, b_head):
    del imag
    adj, xw = _tc_pre(real, graph_sigs, W_conv)
    whead = W_head.reshape(C, N)
    m = whead[:, :, None] * w_pool[:, 0][None, None, :]
    bias2 = b_head + b_pool[0] * jnp.sum(whead, axis=1)
    bias2_p = jnp.concatenate([bias2, jnp.full((L - C,), NEG, jnp.float32)])
    out = _sc_main()(adj, xw, m, b_conv, bias2_p)
    return out[:, :C]


# speedup vs baseline: 1.1973x; 1.1973x over previous
import functools

import jax
import jax.numpy as jnp
from jax import lax
from jax.experimental import pallas as pl
from jax.experimental.pallas import tpu as pltpu
from jax.experimental.pallas import tpu_sc as plsc

B, T, N, F_IN = 256, 8, 30, 128
NF, C = 64, 10
NC, NS = 2, 16
NW = NC * NS
GPW = B // NW
L = 16
NEG = -1e30


GX = 32


def _xw_body(real_ref, wconv_ref, out_ref):
    x = real_ref[...].reshape(GX * N, F_IN)
    xw = jnp.dot(x, wconv_ref[...], preferred_element_type=jnp.float32,
                 precision=lax.Precision.HIGHEST)
    out_ref[...] = (xw * (1.0 / T)).reshape(GX, N, NF)


def _tc_xw(real, W_conv):
    return pl.pallas_call(
        _xw_body,
        grid=(B // GX,),
        in_specs=[pl.BlockSpec((GX, N, F_IN), lambda i: (i, 0, 0)),
                  pl.BlockSpec((F_IN, NF), lambda i: (0, 0))],
        out_specs=pl.BlockSpec((GX, N, NF), lambda i: (i, 0, 0)),
        out_shape=jax.ShapeDtypeStruct((B, N, NF), jnp.float32),
    )(real, W_conv)


_JBLOCKS = ((0, 8), (8, 8), (16, 8), (24, 6))


def _bfly(v, op):
    dnums = lax.GatherDimensionNumbers(offset_dims=(), collapsed_slice_dims=(0,),
                                       start_index_map=(0,))
    lane = lax.iota(jnp.int32, L)
    for s in (8, 4, 2, 1):
        perm = (lane ^ s)[:, None]
        g = lax.gather(v, perm, dnums, (1,),
                       mode=lax.GatherScatterMode.PROMISE_IN_BOUNDS)
        v = op(v, g)
    return v


def _sc_body(sigs_hbm, xw_hbm, m_hbm, bconv_hbm, bias2_hbm, out_hbm,
             sigs_v, xw_v, m_v, bconv_v, bias2_v, out_v, adj_v, agg_v):
    cid = lax.axis_index("c")
    sid = lax.axis_index("s")
    wid = sid * NC + cid

    pltpu.sync_copy(m_hbm, m_v)
    pltpu.sync_copy(bconv_hbm, bconv_v)
    pltpu.sync_copy(bias2_hbm, bias2_v)
    bconv_r = [bconv_v[pl.ds(fv * L, L)] for fv in range(NF // L)]
    bias2_r = bias2_v[...]

    def graph_body(k, _):
        g = wid * GPW + k
        pltpu.sync_copy(sigs_hbm.at[g], sigs_v)
        pltpu.sync_copy(xw_hbm.at[g], xw_v)

        def mean_body(i, carry):
            lo = sigs_v[0, i, pl.ds(0, L)]
            hi = sigs_v[0, i, pl.ds(N - L, L)]
            for t in range(1, T):
                lo = lo + sigs_v[t, i, pl.ds(0, L)]
                hi = hi + sigs_v[t, i, pl.ds(N - L, L)]
            adj_v[i, pl.ds(0, L)] = lo
            adj_v[i, pl.ds(N - L, L)] = hi
            return carry

        lax.fori_loop(0, N, mean_body, 0)

        for (j0, J) in _JBLOCKS:
            jb = min(j0, N - L)

            def agg_body(i, accs, jb=jb, off=j0 - jb, J=J):
                xwr = [xw_v[i, pl.ds(fv * L, L)] for fv in range(NF // L)]
                av = adj_v[i, pl.ds(jb, L)]
                out = []
                for jj in range(J):
                    a = av[off + jj]
                    row = accs[jj]
                    out.append(tuple(row[fv] + a * xwr[fv]
                                     for fv in range(NF // L)))
                return tuple(out)

            init = tuple(tuple(jnp.zeros((L,), jnp.float32)
                               for _ in range(NF // L)) for _ in range(J))
            accs = lax.fori_loop(0, N, agg_body, init)
            for jj in range(J):
                for fv in range(NF // L):
                    agg_v[j0 + jj, pl.ds(fv * L, L)] = accs[jj][fv]

        def head_body(j, accC):
            out = list(accC)
            for fv in range(NF // L):
                h = jnp.maximum(agg_v[j, pl.ds(fv * L, L)] + bconv_r[fv], 0.0)
                for c in range(C):
                    out[c] = out[c] + m_v[c, j, pl.ds(fv * L, L)] * h
            return tuple(out)

        accC = lax.fori_loop(0, N, head_body,
                             tuple(jnp.zeros((L,), jnp.float32)
                                   for _ in range(C)))

        lane = lax.iota(jnp.int32, L)
        lv = bias2_r
        for c in range(C):
            lv = lv + jnp.where(lane == c, _bfly(accC[c], jnp.add), 0.0)
        mxv = _bfly(lv, jnp.maximum)
        e = jnp.exp(lv - mxv)
        sv = _bfly(e, jnp.add)
        out_v[k, :] = e / sv
        return 0

    lax.fori_loop(0, GPW, graph_body, 0)
    pltpu.sync_copy(out_v, out_hbm.at[pl.ds(wid * GPW, GPW)])


@functools.lru_cache(maxsize=1)
def _sc_main():
    return pl.kernel(
        _sc_body,
        out_type=jax.ShapeDtypeStruct((B, L), jnp.float32),
        mesh=plsc.VectorSubcoreMesh(core_axis_name="c", subcore_axis_name="s",
                                    num_cores=NC, num_subcores=NS),
        scratch_types=[
            pltpu.VMEM((T, N, N), jnp.float32),
            pltpu.VMEM((N, NF), jnp.float32),
            pltpu.VMEM((C, N, NF), jnp.float32),
            pltpu.VMEM((NF,), jnp.float32),
            pltpu.VMEM((L,), jnp.float32),
            pltpu.VMEM((GPW, L), jnp.float32),
            pltpu.VMEM((N, N), jnp.float32),
            pltpu.VMEM((N, NF), jnp.float32),
        ],
    )


def kernel(real, imag, graph_sigs, W_conv, b_conv, w_pool, b_pool, W_head, b_head):
    del imag
    xw = _tc_xw(real, W_conv)
    whead = W_head.reshape(C, N)
    m = whead[:, :, None] * w_pool[:, 0][None, None, :]
    bias2 = b_head + b_pool[0] * jnp.sum(whead, axis=1)
    bias2_p = jnp.concatenate([bias2, jnp.full((L - C,), NEG, jnp.float32)])
    out = _sc_main()(graph_sigs, xw, m, b_conv, bias2_p)
    return out[:, :C]


# speedup vs baseline: 1.3750x; 1.1485x over previous
import functools

import jax
import jax.numpy as jnp
from jax import lax
from jax.experimental import pallas as pl
from jax.experimental.pallas import tpu as pltpu
from jax.experimental.pallas import tpu_sc as plsc

B, T, N, F_IN = 256, 8, 30, 128
NF, C = 64, 10
NC, NS = 2, 16
NW = NC * NS
GPW = B // NW
L = 16

NN = N * N
SIG = T * NN
XWF = N * NF
MF = C * N * NF
FP = 128
NEG = -1e30


GX = 32


def _xw_body(real_ref, wconv_ref, out_ref):
    x = real_ref[...].reshape(GX * N, F_IN)
    xw = jnp.dot(x, wconv_ref[...], preferred_element_type=jnp.float32,
                 precision=lax.Precision.HIGHEST)
    out_ref[...] = xw * (1.0 / T)


def _tc_xw(real, wconv_p):
    return pl.pallas_call(
        _xw_body,
        grid=(B // GX,),
        in_specs=[pl.BlockSpec((GX, N, F_IN), lambda i: (i, 0, 0)),
                  pl.BlockSpec((F_IN, FP), lambda i: (0, 0))],
        out_specs=pl.BlockSpec((GX * N, FP), lambda i: (i, 0)),
        out_shape=jax.ShapeDtypeStruct((B * N, FP), jnp.float32),
    )(real, wconv_p)


_JBLOCKS = ((0, 8), (8, 8), (16, 8), (24, 6))


def _bfly(v, op):
    dnums = lax.GatherDimensionNumbers(offset_dims=(), collapsed_slice_dims=(0,),
                                       start_index_map=(0,))
    lane = lax.iota(jnp.int32, L)
    for s in (8, 4, 2, 1):
        perm = (lane ^ s)[:, None]
        g = lax.gather(v, perm, dnums, (1,),
                       mode=lax.GatherScatterMode.PROMISE_IN_BOUNDS)
        v = op(v, g)
    return v


def _sc_body(sigs_hbm, xw_hbm, m_hbm, bconv_hbm, bias2_hbm, out_hbm,
             sigs_v, xw_v, m_v, bconv_v, bias2_v, out_v, adj_v, agg_v):
    cid = lax.axis_index("c")
    sid = lax.axis_index("s")
    wid = sid * NC + cid

    pltpu.sync_copy(m_hbm, m_v)
    pltpu.sync_copy(bconv_hbm, bconv_v)
    pltpu.sync_copy(bias2_hbm, bias2_v)
    bconv_r = [bconv_v[pl.ds(fv * L, L)] for fv in range(NF // L)]
    bias2_r = bias2_v[...]
    pltpu.sync_copy(xw_hbm.at[pl.ds(wid * GPW * N, GPW * N)], xw_v)

    def graph_body(k, _):
        g = wid * GPW + k
        pltpu.sync_copy(sigs_hbm.at[pl.ds(g * SIG, SIG)], sigs_v)

        def mean_body(c2, carry):
            base = jnp.minimum(c2 * L, NN - L)
            acc = sigs_v[pl.ds(base, L)]
            for t in range(1, T):
                acc = acc + sigs_v[pl.ds(t * NN + base, L)]
            adj_v[pl.ds(base, L)] = acc
            return carry

        lax.fori_loop(0, (NN + L - 1) // L, mean_body, 0)

        for (j0, J) in _JBLOCKS:
            def agg_body(i, accs, j0=j0, J=J, k=k):
                xwr = [xw_v[k * N + i, pl.ds(fv * L, L)]
                       for fv in range(NF // L)]
                av = adj_v[pl.ds(i * N + j0, L)]
                out = []
                for jj in range(J):
                    a = av[jj]
                    row = accs[jj]
                    out.append(tuple(row[fv] + a * xwr[fv]
                                     for fv in range(NF // L)))
                return tuple(out)

            init = tuple(tuple(jnp.zeros((L,), jnp.float32)
                               for _ in range(NF // L)) for _ in range(J))
            accs = lax.fori_loop(0, N, agg_body, init)
            for jj in range(J):
                for fv in range(NF // L):
                    agg_v[pl.ds((j0 + jj) * NF + fv * L, L)] = accs[jj][fv]

        def head_body(j, accC):
            jb = j * NF
            out = list(accC)
            for fv in range(NF // L):
                h = jnp.maximum(agg_v[pl.ds(jb + fv * L, L)] + bconv_r[fv], 0.0)
                for c in range(C):
                    out[c] = out[c] + m_v[pl.ds(c * XWF + jb + fv * L, L)] * h
            return tuple(out)

        accC = lax.fori_loop(0, N, head_body,
                             tuple(jnp.zeros((L,), jnp.float32)
                                   for _ in range(C)))

        lane = lax.iota(jnp.int32, L)
        lv = bias2_r
        for c in range(C):
            lv = lv + jnp.where(lane == c, _bfly(accC[c], jnp.add), 0.0)
        mxv = _bfly(lv, jnp.maximum)
        e = jnp.exp(lv - mxv)
        sv = _bfly(e, jnp.add)
        out_v[pl.ds(k * L, L)] = e / sv
        return 0

    lax.fori_loop(0, GPW, graph_body, 0)
    pltpu.sync_copy(out_v, out_hbm.at[pl.ds(wid * GPW * L, GPW * L)])


@functools.lru_cache(maxsize=1)
def _sc_main():
    return pl.kernel(
        _sc_body,
        out_type=jax.ShapeDtypeStruct((B * L,), jnp.float32),
        mesh=plsc.VectorSubcoreMesh(core_axis_name="c", subcore_axis_name="s",
                                    num_cores=NC, num_subcores=NS),
        scratch_types=[
            pltpu.VMEM((SIG,), jnp.float32),
            pltpu.VMEM((GPW * N, FP), jnp.float32),
            pltpu.VMEM((MF,), jnp.float32),
            pltpu.VMEM((NF,), jnp.float32),
            pltpu.VMEM((L,), jnp.float32),
            pltpu.VMEM((GPW * L,), jnp.float32),
            pltpu.VMEM((NN + L,), jnp.float32),
            pltpu.VMEM((XWF,), jnp.float32),
        ],
    )


def kernel(real, imag, graph_sigs, W_conv, b_conv, w_pool, b_pool, W_head, b_head):
    del imag
    wconv_p = jnp.concatenate(
        [W_conv, jnp.zeros((F_IN, FP - NF), jnp.float32)], axis=1)
    xw = _tc_xw(real, wconv_p)
    sigs = graph_sigs.reshape(B * SIG)
    whead = W_head.reshape(C, N)
    m = (whead[:, :, None] * w_pool[:, 0][None, None, :]).reshape(MF)
    bias2 = b_head + b_pool[0] * jnp.sum(whead, axis=1)
    bias2_p = jnp.concatenate([bias2, jnp.full((L - C,), NEG, jnp.float32)])
    out = _sc_main()(sigs, xw, m, b_conv, bias2_p)
    return out.reshape(B, L)[:, :C]


# speedup vs baseline: 1.4822x; 1.0780x over previous
import functools

import jax
import jax.numpy as jnp
from jax import lax
from jax.experimental import pallas as pl
from jax.experimental.pallas import tpu as pltpu
from jax.experimental.pallas import tpu_sc as plsc

B, T, N, F_IN = 256, 8, 30, 128
NF, C = 64, 10
NC, NS = 2, 16
NW = NC * NS
GPW = B // NW
L = 16

NN = N * N
SIG = T * NN
XWF = N * NF
MF = C * N * NF
FP = 128
NEG = -1e30


GX = 32


def _xw_body(real_ref, wconv_ref, out_ref):
    x = real_ref[...].reshape(GX * N, F_IN)
    xw = jnp.dot(x, wconv_ref[...], preferred_element_type=jnp.float32)
    out_ref[...] = xw * (1.0 / T)


def _tc_xw(real, wconv_p):
    return pl.pallas_call(
        _xw_body,
        grid=(B // GX,),
        in_specs=[pl.BlockSpec((GX, N, F_IN), lambda i: (i, 0, 0)),
                  pl.BlockSpec((F_IN, FP), lambda i: (0, 0))],
        out_specs=pl.BlockSpec((GX * N, FP), lambda i: (i, 0)),
        out_shape=jax.ShapeDtypeStruct((B * N, FP), jnp.float32),
    )(real, wconv_p)


_JBLOCKS = ((0, 8), (8, 8), (16, 8), (24, 6))


def _bfly(v, op):
    dnums = lax.GatherDimensionNumbers(offset_dims=(), collapsed_slice_dims=(0,),
                                       start_index_map=(0,))
    lane = lax.iota(jnp.int32, L)
    for s in (8, 4, 2, 1):
        perm = (lane ^ s)[:, None]
        g = lax.gather(v, perm, dnums, (1,),
                       mode=lax.GatherScatterMode.PROMISE_IN_BOUNDS)
        v = op(v, g)
    return v


def _sc_body(sigs_hbm, xw_hbm, m_hbm, bconv_hbm, bias2_hbm, out_hbm,
             sigs_v, xw_v, m_v, bconv_v, bias2_v, out_v, adj_v, agg_v,
             sem0, sem1):
    cid = lax.axis_index("c")
    sid = lax.axis_index("s")
    wid = sid * NC + cid

    pltpu.sync_copy(m_hbm, m_v)
    pltpu.sync_copy(bconv_hbm, bconv_v)
    pltpu.sync_copy(bias2_hbm, bias2_v)
    bconv_r = [bconv_v[pl.ds(fv * L, L)] for fv in range(NF // L)]
    bias2_r = bias2_v[...]
    pltpu.sync_copy(xw_hbm.at[pl.ds(wid * GPW * N, GPW * N)], xw_v)

    def sigs_dma(g, p, sem):
        return pltpu.make_async_copy(
            sigs_hbm.at[pl.ds(g * SIG, SIG)],
            sigs_v.at[pl.ds(p * SIG, SIG)], sem)

    g_last = wid * GPW + GPW - 1
    sigs_dma(wid * GPW, 0, sem0).start()

    def pair_body(k2, _):
        for half in range(2):
            k = k2 * 2 + half
            g = wid * GPW + k
            nxt = 1 - half
            sem_n = sem1 if nxt else sem0
            sem_c = sem1 if half else sem0
            gn = jnp.minimum(g + 1, g_last)
            sigs_dma(gn, nxt, sem_n).start()
            sigs_dma(g, half, sem_c).wait()
            sb = half * SIG

            def mean_body(c2, carry, sb=sb):
                base = jnp.minimum(c2 * L, NN - L)
                acc = sigs_v[pl.ds(sb + base, L)]
                for t in range(1, T):
                    acc = acc + sigs_v[pl.ds(sb + t * NN + base, L)]
                adj_v[pl.ds(base, L)] = acc
                return carry

            lax.fori_loop(0, (NN + L - 1) // L, mean_body, 0)

            for (j0, J) in _JBLOCKS:
                def agg_body(i, accs, j0=j0, J=J, k=k):
                    xwr = [xw_v[k * N + i, pl.ds(fv * L, L)]
                           for fv in range(NF // L)]
                    av = adj_v[pl.ds(i * N + j0, L)]
                    out = []
                    for jj in range(J):
                        a = av[jj]
                        row = accs[jj]
                        out.append(tuple(row[fv] + a * xwr[fv]
                                         for fv in range(NF // L)))
                    return tuple(out)

                init = tuple(tuple(jnp.zeros((L,), jnp.float32)
                                   for _ in range(NF // L)) for _ in range(J))
                accs = lax.fori_loop(0, N, agg_body, init)
                for jj in range(J):
                    for fv in range(NF // L):
                        agg_v[pl.ds(half * XWF + (j0 + jj) * NF + fv * L, L)] \
                            = accs[jj][fv]

        def head_body(j, accC):
            jb = j * NF
            o0, o1 = list(accC[0]), list(accC[1])
            for fv in range(NF // L):
                h0 = jnp.maximum(agg_v[pl.ds(jb + fv * L, L)]
                                 + bconv_r[fv], 0.0)
                h1 = jnp.maximum(agg_v[pl.ds(XWF + jb + fv * L, L)]
                                 + bconv_r[fv], 0.0)
                for c in range(C):
                    mv = m_v[pl.ds(c * XWF + jb + fv * L, L)]
                    o0[c] = o0[c] + mv * h0
                    o1[c] = o1[c] + mv * h1
            return (tuple(o0), tuple(o1))

        zc = tuple(jnp.zeros((L,), jnp.float32) for _ in range(C))
        accC = lax.fori_loop(0, N, head_body, (zc, zc))

        lane = lax.iota(jnp.int32, L)
        for half in range(2):
            lv = bias2_r
            for c in range(C):
                lv = lv + jnp.where(lane == c, _bfly(accC[half][c], jnp.add),
                                    0.0)
            mxv = _bfly(lv, jnp.maximum)
            e = jnp.exp(lv - mxv)
            sv = _bfly(e, jnp.add)
            out_v[pl.ds((k2 * 2 + half) * L, L)] = e / sv
        return 0

    lax.fori_loop(0, GPW // 2, pair_body, 0)
    sigs_dma(g_last, 0, sem0).wait()
    pltpu.sync_copy(out_v, out_hbm.at[pl.ds(wid * GPW * L, GPW * L)])


@functools.lru_cache(maxsize=1)
def _sc_main():
    return pl.kernel(
        _sc_body,
        out_type=jax.ShapeDtypeStruct((B * L,), jnp.float32),
        mesh=plsc.VectorSubcoreMesh(core_axis_name="c", subcore_axis_name="s",
                                    num_cores=NC, num_subcores=NS),
        scratch_types=[
            pltpu.VMEM((2 * SIG,), jnp.float32),
            pltpu.VMEM((GPW * N, FP), jnp.float32),
            pltpu.VMEM((MF,), jnp.float32),
            pltpu.VMEM((NF,), jnp.float32),
            pltpu.VMEM((L,), jnp.float32),
            pltpu.VMEM((GPW * L,), jnp.float32),
            pltpu.VMEM((NN + L,), jnp.float32),
            pltpu.VMEM((2 * XWF,), jnp.float32),
            pltpu.SemaphoreType.DMA,
            pltpu.SemaphoreType.DMA,
        ],
    )


def kernel(real, imag, graph_sigs, W_conv, b_conv, w_pool, b_pool, W_head, b_head):
    del imag
    wconv_p = jnp.concatenate(
        [W_conv, jnp.zeros((F_IN, FP - NF), jnp.float32)], axis=1)
    xw = _tc_xw(real, wconv_p)
    sigs = graph_sigs.reshape(B * SIG)
    whead = W_head.reshape(C, N)
    m = (whead[:, :, None] * w_pool[:, 0][None, None, :]).reshape(MF)
    bias2 = b_head + b_pool[0] * jnp.sum(whead, axis=1)
    bias2_p = jnp.concatenate([bias2, jnp.full((L - C,), NEG, jnp.float32)])
    out = _sc_main()(sigs, xw, m, b_conv, bias2_p)
    return out.reshape(B, L)[:, :C]
